# Initial kernel scaffold; baseline (speedup 1.0000x reference)
import jax
import jax.numpy as jnp
from jax.experimental import pallas as pl


def kernel(x, edge_index, W1, b1, W2, b2):
    raise NotImplementedError("write your pallas kernel here")


# speedup vs baseline: 11.9394x; 11.9394x over previous
import functools

import jax
import jax.numpy as jnp
from jax import lax
from jax.experimental import pallas as pl
from jax.experimental.pallas import tpu as pltpu
from jax.experimental.pallas import tpu_sc as plsc

NC = 2
NS = 16
CHUNK = 80

@functools.cache
def _mesh():
    return plsc.VectorSubcoreMesh(core_axis_name="c", subcore_axis_name="s",
                                  num_cores=NC, num_subcores=NS)


def _deg_body(n, ept, nch, ones_hbm, row_hbm, zeros_hbm, out_hbm,
              idx_v, ones_v, acc_sh, sem):
    c = lax.axis_index("c")
    s = lax.axis_index("s")
    rpt = n // NS
    pltpu.sync_copy(zeros_hbm.at[pl.ds(s * rpt, rpt)],
                    acc_sh.at[pl.ds(s * rpt, rpt)])
    pltpu.sync_copy(ones_hbm, ones_v)
    plsc.subcore_barrier()
    base = (c * NS + s) * ept

    def chunk(i, carry):
        pltpu.sync_copy(row_hbm.at[pl.ds(base + i * CHUNK, CHUNK)], idx_v)
        pltpu.sync_copy(ones_v, acc_sh.at[idx_v], add=True)
        return carry

    lax.fori_loop(0, nch, chunk, 0)
    plsc.subcore_barrier()
    pltpu.sync_copy(acc_sh.at[pl.ds(s * rpt, rpt)],
                    out_hbm.at[c, pl.ds(s * rpt, rpt)])


def _agg_body(n, d, ept, nch, s_hbm, row_hbm, col_hbm, zeros_hbm, out_hbm,
              ridx_v, cidx_v, rows_v, acc_sh, sem):
    c = lax.axis_index("c")
    s = lax.axis_index("s")
    rpt = n // NS
    pltpu.sync_copy(zeros_hbm.at[pl.ds(s * rpt, rpt)],
                    acc_sh.at[pl.ds(s * rpt, rpt)])
    plsc.subcore_barrier()
    base = (c * NS + s) * ept

    def chunk(i, carry):
        off = base + i * CHUNK
        pltpu.sync_copy(row_hbm.at[pl.ds(off, CHUNK)], ridx_v)
        pltpu.sync_copy(col_hbm.at[pl.ds(off, CHUNK)], cidx_v)
        pltpu.async_copy(s_hbm.at[ridx_v], rows_v, sem).wait()
        pltpu.sync_copy(rows_v, acc_sh.at[cidx_v], add=True)
        return carry

    lax.fori_loop(0, nch, chunk, 0)
    plsc.subcore_barrier()
    pltpu.sync_copy(acc_sh.at[pl.ds(s * rpt, rpt)],
                    out_hbm.at[c, pl.ds(s * rpt, rpt)])


def _pad_rows(n):
    rpt = -(-n // NS)
    rpt = -(-rpt // 8) * 8
    return rpt * NS


def _sc_degree(row, n):
    e = row.shape[0]
    ept = e // (NC * NS)
    nch = ept // CHUNK
    npad = _pad_rows(n)
    ones = jnp.ones((CHUNK, 128), jnp.float32)
    zeros = jnp.zeros((npad, 128), jnp.float32)
    body = functools.partial(_deg_body, npad, ept, nch)
    return pl.kernel(
        body,
        out_type=jax.ShapeDtypeStruct((NC, npad, 128), jnp.float32),
        mesh=_mesh(),
        scratch_types=[
            pltpu.VMEM((CHUNK,), jnp.int32),
            pltpu.VMEM((CHUNK, 128), jnp.float32),
            pltpu.VMEM_SHARED((npad, 128), jnp.float32),
            pltpu.SemaphoreType.DMA,
        ],
    )(ones, row, zeros)


def _sc_aggregate(sval, row, col):
    n, d = sval.shape
    e = row.shape[0]
    ept = e // (NC * NS)
    nch = ept // CHUNK
    npad = _pad_rows(n)
    zeros = jnp.zeros((npad, d), jnp.float32)
    body = functools.partial(_agg_body, npad, d, ept, nch)
    return pl.kernel(
        body,
        out_type=jax.ShapeDtypeStruct((NC, npad, d), jnp.float32),
        mesh=_mesh(),
        scratch_types=[
            pltpu.VMEM((CHUNK,), jnp.int32),
            pltpu.VMEM((CHUNK,), jnp.int32),
            pltpu.VMEM((CHUNK, d), jnp.float32),
            pltpu.VMEM_SHARED((npad, d), jnp.float32),
            pltpu.SemaphoreType.DMA,
        ],
    )(sval, row, col, zeros)


def _dis_from(dp):
    deg = (dp[0] + dp[1]).sum(axis=-1) * (1.0 / 128.0) + 1.0
    return lax.rsqrt(deg)


def _tc1_body(x_ref, w_ref, dp_ref, o_ref):
    dis = _dis_from(dp_ref[...])
    y = lax.dot_general(x_ref[...], w_ref[...], (((1,), (1,)), ((), ())),
                        preferred_element_type=jnp.float32)
    o_ref[...] = y * dis[:, None]


def _tc2_body(agg_ref, s1_ref, dp_ref, b1_ref, w2_ref, o_ref):
    dis = _dis_from(dp_ref[...])
    t = (agg_ref[0] + agg_ref[1] + s1_ref[...]) * dis[:, None] + b1_ref[...]
    h = jnp.maximum(t, 0.0)
    y = lax.dot_general(h, w2_ref[...], (((1,), (1,)), ((), ())),
                        preferred_element_type=jnp.float32)
    o_ref[...] = y * dis[:, None]


def _tc3_body(agg_ref, s2_ref, dp_ref, b2_ref, o_ref):
    dis = _dis_from(dp_ref[...])
    o = (agg_ref[0] + agg_ref[1] + s2_ref[...]) * dis[:, None] + b2_ref[...]
    m = jnp.max(o, axis=1, keepdims=True)
    lse = jnp.log(jnp.sum(jnp.exp(o - m), axis=1, keepdims=True)) + m
    o_ref[...] = o - lse


_BR = 400


def _tc_call(body, out_shape, specs, args):
    n = out_shape[0]
    return pl.pallas_call(
        body,
        grid=(n // _BR,),
        in_specs=specs,
        out_specs=pl.BlockSpec((_BR, out_shape[1]), lambda i: (i, 0)),
        out_shape=jax.ShapeDtypeStruct(out_shape, jnp.float32),
    )(*args)


def _spec_rows(d):
    return pl.BlockSpec((_BR, d), lambda i: (i, 0))


def _spec_full(shape):
    nd = len(shape)
    return pl.BlockSpec(shape, lambda i, _nd=nd: (0,) * _nd)


def _spec_part(d):
    return pl.BlockSpec((NC, _BR, d), lambda i: (0, i, 0))


def kernel(x, edge_index, W1, b1, W2, b2):
    n, d_in = x.shape
    row = edge_index[0]
    col = edge_index[1]

    deg_part = _sc_degree(row, n)

    s1 = _tc_call(_tc1_body, (n, W1.shape[0]),
                  [_spec_rows(d_in), _spec_full(W1.shape), _spec_part(128)],
                  (x, W1, deg_part))

    agg1 = _sc_aggregate(s1, row, col)

    s2 = _tc_call(_tc2_body, (n, W2.shape[0]),
                  [_spec_part(s1.shape[1]), _spec_rows(s1.shape[1]),
                   _spec_part(128), _spec_full((1, b1.shape[0])),
                   _spec_full(W2.shape)],
                  (agg1, s1, deg_part, b1.reshape(1, -1), W2))

    agg2 = _sc_aggregate(s2, row, col)

    out = _tc_call(_tc3_body, (n, W2.shape[0]),
                   [_spec_part(s2.shape[1]), _spec_rows(s2.shape[1]),
                    _spec_part(128), _spec_full((1, b2.shape[0]))],
                   (agg2, s2, deg_part, b2.reshape(1, -1)))
    return out
